# Initial kernel scaffold; baseline (speedup 1.0000x reference)
import jax
import jax.numpy as jnp
from jax.experimental import pallas as pl


def kernel(x, lut):
    raise NotImplementedError("write your pallas kernel here")


# speedup vs baseline: 1.4770x; 1.4770x over previous
import functools
import math

import jax
import jax.numpy as jnp
from jax import lax
from jax.experimental import pallas as pl
from jax.experimental.pallas import tpu as pltpu
from jax.experimental.pallas import tpu_sc as plsc

D_M = 1024
NC, NS, L = 2, 16, 16
NW = NC * NS
B_TOT = 4 * 2048
B_PER_W = B_TOT // NW
C = 32
NCHUNK = B_PER_W // C
SCALE = math.sqrt(float(D_M))

_mesh = plsc.VectorSubcoreMesh(
    core_axis_name="c", subcore_axis_name="s", num_cores=NC, num_subcores=NS
)


@functools.partial(
    pl.kernel,
    out_type=jax.ShapeDtypeStruct((B_TOT, D_M), jnp.float32),
    mesh=_mesh,
    scratch_types=[
        pltpu.VMEM((NCHUNK, C), jnp.int32),
        pltpu.VMEM((C, D_M), jnp.float32),
        pltpu.VMEM((C, D_M), jnp.float32),
        pltpu.SemaphoreType.DMA,
        pltpu.SemaphoreType.DMA,
        pltpu.SemaphoreType.DMA,
        pltpu.SemaphoreType.DMA,
    ],
)
def _emb_lookup(x_hbm, lut_hbm, out_hbm, idx_v, buf0, buf1,
                gsem0, gsem1, ssem0, ssem1):
    wid = lax.axis_index("s") * NC + lax.axis_index("c")
    base = wid * B_PER_W
    bufs = (buf0, buf1)
    gsems = (gsem0, gsem1)
    ssems = (ssem0, ssem1)

    pltpu.sync_copy(x_hbm.at[wid], idx_v)

    def start_gather(g):
        b = g & 1
        return pltpu.async_copy(lut_hbm.at[idx_v.at[g]], bufs[b], gsems[b])

    gd = [None] * NCHUNK
    sd = [None] * NCHUNK
    gd[0] = start_gather(0)
    for g in range(NCHUNK):
        b = g & 1
        gd[g].wait()
        if g + 1 < NCHUNK:
            if g >= 1:
                sd[g - 1].wait()
            gd[g + 1] = start_gather(g + 1)
        buf = bufs[b]

        @pl.loop(0, C, unroll=1)
        def _rows(r, buf=buf):
            @pl.loop(0, D_M // L, unroll=8)
            def _cols(cidx, r=r, buf=buf):
                sl = pl.ds(cidx * L, L)
                buf[r, sl] = buf[r, sl] * SCALE

        sd[g] = pltpu.async_copy(
            buf, out_hbm.at[pl.ds(base + g * C, C)], ssems[b]
        )
    sd[NCHUNK - 2].wait()
    sd[NCHUNK - 1].wait()


def kernel(x, lut):
    xr = x.reshape(NW, NCHUNK, C).astype(jnp.int32)
    out = _emb_lookup(xr, lut)
    return out.reshape(x.shape + (lut.shape[1],))
